# baseline (speedup 1.0000x reference)
import functools

import jax
import jax.numpy as jnp
from jax import lax
from jax.experimental import pallas as pl
from jax.experimental.pallas import tpu as pltpu
from jax.experimental.pallas import tpu_sc as plsc

N = 10000
D = 256
HALF = 128
E = 160000
NSC = 2
NTILES = 16
K = 96
NBUF = 3
LOOKAHEAD = NBUF - 1
EPT = E // NTILES
NCH = -(-EPT // K)
PADE = NCH * K
NPADROW = 8
RPT = 624
REM = N - RPT * NTILES

assert EPT * NTILES == E and REM % 8 == 0 and REM >= 0


def _mm_body(h_ref, w_ref, b_ref, y_ref, z_ref, *, relu_in, split_in):
    if split_in:
        h0 = h_ref[0]
        h1 = h_ref[1]
        if relu_in:
            h0 = jnp.maximum(h0, 0.0)
            h1 = jnp.maximum(h1, 0.0)
        out = jnp.dot(h0, w_ref[0:HALF, :], preferred_element_type=jnp.float32)
        out = out + jnp.dot(h1, w_ref[HALF:D, :], preferred_element_type=jnp.float32)
    else:
        h = h_ref[...]
        if relu_in:
            h = jnp.maximum(h, 0.0)
        out = jnp.dot(h, w_ref[...], preferred_element_type=jnp.float32)
    y_ref[0] = out[:, 0:HALF]
    y_ref[1] = out[:, HALF:D]
    z_ref[0] = out[:, D:D + HALF] + b_ref[0, 0:HALF]
    z_ref[1] = out[:, D + HALF:2 * D] + b_ref[0, HALF:D]


def _tc_matmul(h, wcat, brel, *, relu_in, split_in, bm=1000):
    grid = (N // bm,)
    if split_in:
        in_spec = pl.BlockSpec((NSC, bm, HALF), lambda i: (0, i, 0))
    else:
        in_spec = pl.BlockSpec((bm, D), lambda i: (i, 0))
    out_spec = pl.BlockSpec((NSC, bm, HALF), lambda i: (0, i, 0))
    return pl.pallas_call(
        functools.partial(_mm_body, relu_in=relu_in, split_in=split_in),
        grid=grid,
        in_specs=[
            in_spec,
            pl.BlockSpec((D, 2 * D), lambda i: (0, 0)),
            pl.BlockSpec((1, D), lambda i: (0, 0)),
        ],
        out_specs=[out_spec, out_spec],
        out_shape=[
            jax.ShapeDtypeStruct((NSC, N, HALF), jnp.float32),
            jax.ShapeDtypeStruct((NSC, N, HALF), jnp.float32),
        ],
    )(h, wcat, brel)


_SC_MESH = plsc.VectorSubcoreMesh(core_axis_name="c", subcore_axis_name="s")


@functools.partial(
    pl.kernel,
    out_type=jax.ShapeDtypeStruct((NSC, N, HALF), jnp.float32),
    mesh=_SC_MESH,
    scratch_types=[
        pltpu.VMEM((PADE,), jnp.int32),
        pltpu.VMEM((NBUF, K), jnp.int32),
        pltpu.VMEM((NBUF, K), jnp.int32),
        pltpu.VMEM((NBUF, K, HALF), jnp.float32),
        pltpu.VMEM_SHARED((N + NPADROW, HALF), jnp.float32),
        pltpu.SemaphoreType.DMA((NBUF,)),
        pltpu.SemaphoreType.DMA((NBUF,)),
    ],
)
def _sc_aggregate(y_hbm, z_hbm, packed_hbm, out_hbm,
                  packed_v, src_ring, dst_ring, rows_v, acc, sem, sem_s):
    c = lax.axis_index("c")
    s = lax.axis_index("s")
    pltpu.sync_copy(z_hbm.at[c, pl.ds(s * RPT, RPT)], acc.at[pl.ds(s * RPT, RPT)])

    @pl.when(s == NTILES - 1)
    def _():
        pltpu.sync_copy(z_hbm.at[c, pl.ds(RPT * NTILES, REM)],
                        acc.at[pl.ds(RPT * NTILES, REM)])

    pltpu.sync_copy(packed_hbm.at[s], packed_v)
    plsc.subcore_barrier()

    def launch(j):
        b = lax.rem(j, NBUF)
        for t in range(K // 16):
            v = packed_v[pl.ds(j * K + t * 16, 16)]
            src_ring[b, pl.ds(t * 16, 16)] = jnp.bitwise_and(v, jnp.int32(0xFFFF))
            dst_ring[b, pl.ds(t * 16, 16)] = lax.shift_right_logical(v, jnp.int32(16))
        pltpu.async_copy(y_hbm.at[c].at[src_ring.at[b]], rows_v.at[b], sem.at[b])

    def wait_scatter(j):
        b = lax.rem(j, NBUF)
        pltpu.make_async_copy(rows_v.at[b], acc.at[dst_ring.at[b]],
                              sem_s.at[b]).wait()

    for j0 in range(LOOKAHEAD):
        launch(jnp.int32(j0))

    def body(j, carry):
        @pl.when(j >= NBUF - LOOKAHEAD)
        def _():
            wait_scatter(j - (NBUF - LOOKAHEAD))

        @pl.when(j + LOOKAHEAD < NCH)
        def _():
            launch(j + LOOKAHEAD)
        b = lax.rem(j, NBUF)
        pltpu.make_async_copy(y_hbm.at[c].at[src_ring.at[b]],
                              rows_v.at[b], sem.at[b]).wait()
        pltpu.async_copy(rows_v.at[b], acc.at[dst_ring.at[b]], sem_s.at[b],
                         add=True)
        return carry

    lax.fori_loop(0, NCH, body, 0)
    for d in range(NBUF - LOOKAHEAD):
        wait_scatter(jnp.int32(NCH - 1 - d))
    plsc.subcore_barrier()
    pltpu.sync_copy(acc.at[pl.ds(s * RPT, RPT)], out_hbm.at[c, pl.ds(s * RPT, RPT)])

    @pl.when(s == NTILES - 1)
    def _():
        pltpu.sync_copy(acc.at[pl.ds(RPT * NTILES, REM)],
                        out_hbm.at[c, pl.ds(RPT * NTILES, REM)])


def _pool_body(h_ref, wm_ref, bm_ref, o_ref):
    h0 = jnp.maximum(h_ref[0], 0.0)
    h1 = jnp.maximum(h_ref[1], 0.0)
    p0 = jnp.max(h0, axis=0, keepdims=True)
    p1 = jnp.max(h1, axis=0, keepdims=True)
    pooled = jnp.concatenate([p0, p1], axis=1)
    o_ref[...] = jnp.dot(pooled, wm_ref[...],
                         preferred_element_type=jnp.float32) + bm_ref[...]


def _tc_pool_mlp(h, wmlp_t, bmlp):
    return pl.pallas_call(
        _pool_body,
        out_shape=jax.ShapeDtypeStruct((1, 10), jnp.float32),
    )(h, wmlp_t, bmlp)


def kernel(x, edge_index, Wrel0, brel0, Wroot0, Wrel1, brel1, Wroot1,
           Wrel2, brel2, Wroot2, Wmlp, bmlp):
    src = edge_index[0].reshape(NTILES, EPT)
    dst = edge_index[1].reshape(NTILES, EPT)
    npad = PADE - EPT
    pad_src = jnp.zeros((NTILES, npad), dtype=jnp.int32)
    pad_dst = jnp.broadcast_to(
        N + (jnp.arange(npad, dtype=jnp.int32) % NPADROW), (NTILES, npad))
    src = jnp.concatenate([src, pad_src], axis=1)
    dst = jnp.concatenate([dst, pad_dst], axis=1)
    packed = ((dst << 16) | src).reshape(NTILES, PADE)

    def layer(h, Wrel, brel, Wroot, relu_in, split_in):
        wcat = jnp.concatenate([Wrel.T, Wroot.T], axis=1)
        y, z = _tc_matmul(h, wcat, brel.reshape(1, D),
                          relu_in=relu_in, split_in=split_in)
        return _sc_aggregate(y, z, packed)

    h1 = layer(x, Wrel0, brel0, Wroot0, relu_in=False, split_in=False)
    h2 = layer(h1, Wrel1, brel1, Wroot1, relu_in=True, split_in=True)
    h3 = layer(h2, Wrel2, brel2, Wroot2, relu_in=True, split_in=True)
    return _tc_pool_mlp(h3, Wmlp.T, bmlp.reshape(1, 10))


# speedup vs baseline: 1.3704x; 1.3704x over previous
import functools

import jax
import jax.numpy as jnp
from jax import lax
from jax.experimental import pallas as pl
from jax.experimental.pallas import tpu as pltpu
from jax.experimental.pallas import tpu_sc as plsc

N = 10000
D = 256
HALF = 128
E = 160000
NSC = 2
NTILES = 16
K = 80
NBUF = 3
LOOKAHEAD = NBUF - 1
EPT = E // NTILES
NCH = -(-EPT // K)
PADE = NCH * K
NPADROW = 8
RPT = 624
REM = N - RPT * NTILES

assert EPT * NTILES == E and REM % 8 == 0 and REM >= 0


def _mm_body(h_ref, w_ref, b_ref, y_ref, z_ref, *, relu_in, split_in):
    if split_in:
        h0 = h_ref[0]
        h1 = h_ref[1]
        if relu_in:
            h0 = jnp.maximum(h0, 0.0)
            h1 = jnp.maximum(h1, 0.0)
        out = jnp.dot(h0, w_ref[0:HALF, :], preferred_element_type=jnp.float32)
        out = out + jnp.dot(h1, w_ref[HALF:D, :], preferred_element_type=jnp.float32)
    else:
        h = h_ref[...]
        if relu_in:
            h = jnp.maximum(h, 0.0)
        out = jnp.dot(h, w_ref[...], preferred_element_type=jnp.float32)
    y_ref[0] = out[:, 0:HALF]
    y_ref[1] = out[:, HALF:D]
    z_ref[0] = out[:, D:D + HALF] + b_ref[0, 0:HALF]
    z_ref[1] = out[:, D + HALF:2 * D] + b_ref[0, HALF:D]


def _tc_matmul(h, wcat, brel, *, relu_in, split_in, bm=1000):
    grid = (N // bm,)
    if split_in:
        in_spec = pl.BlockSpec((NSC, bm, HALF), lambda i: (0, i, 0))
    else:
        in_spec = pl.BlockSpec((bm, D), lambda i: (i, 0))
    out_spec = pl.BlockSpec((NSC, bm, HALF), lambda i: (0, i, 0))
    return pl.pallas_call(
        functools.partial(_mm_body, relu_in=relu_in, split_in=split_in),
        grid=grid,
        in_specs=[
            in_spec,
            pl.BlockSpec((D, 2 * D), lambda i: (0, 0)),
            pl.BlockSpec((1, D), lambda i: (0, 0)),
        ],
        out_specs=[out_spec, out_spec],
        out_shape=[
            jax.ShapeDtypeStruct((NSC, N, HALF), jnp.float32),
            jax.ShapeDtypeStruct((NSC, N, HALF), jnp.float32),
        ],
    )(h, wcat, brel)


_SC_MESH = plsc.VectorSubcoreMesh(core_axis_name="c", subcore_axis_name="s")


@functools.partial(
    pl.kernel,
    out_type=jax.ShapeDtypeStruct((NSC, N, HALF), jnp.float32),
    mesh=_SC_MESH,
    scratch_types=[
        pltpu.VMEM((NCH, K), jnp.int32),
        pltpu.VMEM((NBUF, K), jnp.int32),
        pltpu.VMEM((NBUF, K), jnp.int32),
        pltpu.VMEM((NBUF, K, HALF), jnp.float32),
        pltpu.VMEM_SHARED((N + NPADROW, HALF), jnp.float32),
        pltpu.SemaphoreType.DMA((NBUF,)),
        pltpu.SemaphoreType.DMA((NBUF,)),
    ],
)
def _sc_aggregate(y_hbm, z_hbm, packed_hbm, out_hbm,
                  packed_v, src_ring, dst_ring, rows_v, acc, sem, sem_s):
    c = lax.axis_index("c")
    s = lax.axis_index("s")
    pltpu.sync_copy(z_hbm.at[c, pl.ds(s * RPT, RPT)], acc.at[pl.ds(s * RPT, RPT)])

    @pl.when(s == NTILES - 1)
    def _():
        pltpu.sync_copy(z_hbm.at[c, pl.ds(RPT * NTILES, REM)],
                        acc.at[pl.ds(RPT * NTILES, REM)])

    pltpu.sync_copy(packed_hbm.at[s], packed_v)
    plsc.subcore_barrier()

    def launch(j):
        b = lax.rem(j, NBUF)
        for t in range(K // 16):
            v = packed_v[j, pl.ds(t * 16, 16)]
            src_ring[b, pl.ds(t * 16, 16)] = jnp.bitwise_and(v, jnp.int32(0xFFFF))
            dst_ring[b, pl.ds(t * 16, 16)] = lax.shift_right_logical(v, jnp.int32(16))
        pltpu.async_copy(y_hbm.at[c].at[src_ring.at[b]], rows_v.at[b], sem.at[b])

    def wait_scatter(j):
        b = lax.rem(j, NBUF)
        pltpu.make_async_copy(rows_v.at[b], acc.at[dst_ring.at[b]],
                              sem_s.at[b]).wait()

    for j0 in range(LOOKAHEAD):
        launch(jnp.int32(j0))

    def body(j, carry):
        @pl.when(j >= NBUF - LOOKAHEAD)
        def _():
            wait_scatter(j - (NBUF - LOOKAHEAD))

        @pl.when(j + LOOKAHEAD < NCH)
        def _():
            launch(j + LOOKAHEAD)
        b = lax.rem(j, NBUF)
        pltpu.make_async_copy(y_hbm.at[c].at[src_ring.at[b]],
                              rows_v.at[b], sem.at[b]).wait()
        pltpu.async_copy(rows_v.at[b], acc.at[dst_ring.at[b]], sem_s.at[b],
                         add=True)
        return carry

    lax.fori_loop(0, NCH, body, 0)
    for d in range(NBUF - LOOKAHEAD):
        wait_scatter(jnp.int32(NCH - 1 - d))
    plsc.subcore_barrier()
    pltpu.sync_copy(acc.at[pl.ds(s * RPT, RPT)], out_hbm.at[c, pl.ds(s * RPT, RPT)])

    @pl.when(s == NTILES - 1)
    def _():
        pltpu.sync_copy(acc.at[pl.ds(RPT * NTILES, REM)],
                        out_hbm.at[c, pl.ds(RPT * NTILES, REM)])


def _pool_body(h_ref, wm_ref, bm_ref, o_ref):
    h0 = jnp.maximum(h_ref[0], 0.0)
    h1 = jnp.maximum(h_ref[1], 0.0)
    p0 = jnp.max(h0, axis=0, keepdims=True)
    p1 = jnp.max(h1, axis=0, keepdims=True)
    pooled = jnp.concatenate([p0, p1], axis=1)
    o_ref[...] = jnp.dot(pooled, wm_ref[...],
                         preferred_element_type=jnp.float32) + bm_ref[...]


def _tc_pool_mlp(h, wmlp_t, bmlp):
    return pl.pallas_call(
        _pool_body,
        out_shape=jax.ShapeDtypeStruct((1, 10), jnp.float32),
    )(h, wmlp_t, bmlp)


def kernel(x, edge_index, Wrel0, brel0, Wroot0, Wrel1, brel1, Wroot1,
           Wrel2, brel2, Wroot2, Wmlp, bmlp):
    src = edge_index[0].reshape(NTILES, EPT)
    dst = edge_index[1].reshape(NTILES, EPT)
    npad = PADE - EPT
    pad_src = jnp.zeros((NTILES, npad), dtype=jnp.int32)
    pad_dst = jnp.broadcast_to(
        N + (jnp.arange(npad, dtype=jnp.int32) % NPADROW), (NTILES, npad))
    src = jnp.concatenate([src, pad_src], axis=1)
    dst = jnp.concatenate([dst, pad_dst], axis=1)
    packed = ((dst << 16) | src).reshape(NTILES, NCH, K)

    def layer(h, Wrel, brel, Wroot, relu_in, split_in):
        wcat = jnp.concatenate([Wrel.T, Wroot.T], axis=1)
        y, z = _tc_matmul(h, wcat, brel.reshape(1, D),
                          relu_in=relu_in, split_in=split_in)
        return _sc_aggregate(y, z, packed)

    h1 = layer(x, Wrel0, brel0, Wroot0, relu_in=False, split_in=False)
    h2 = layer(h1, Wrel1, brel1, Wroot1, relu_in=True, split_in=True)
    h3 = layer(h2, Wrel2, brel2, Wroot2, relu_in=True, split_in=True)
    return _tc_pool_mlp(h3, Wmlp.T, bmlp.reshape(1, 10))


# speedup vs baseline: 1.4043x; 1.0247x over previous
import functools

import jax
import jax.numpy as jnp
from jax import lax
from jax.experimental import pallas as pl
from jax.experimental.pallas import tpu as pltpu
from jax.experimental.pallas import tpu_sc as plsc

N = 10000
D = 256
HALF = 128
E = 160000
NSC = 2
NTILES = 16
K = 80
NBUF = 3
LOOKAHEAD = NBUF - 1
EPT = E // NTILES
NCH = -(-EPT // K)
PADE = NCH * K
NPADROW = 8
RPT = 624
REM = N - RPT * NTILES

assert EPT * NTILES == E and REM % 8 == 0 and REM >= 0


def _mm_body(h_ref, w_ref, b_ref, y_ref, z_ref, *, relu_in, split_in):
    if split_in:
        h0 = h_ref[0]
        h1 = h_ref[1]
        if relu_in:
            h0 = jnp.maximum(h0, 0.0)
            h1 = jnp.maximum(h1, 0.0)
        out = jnp.dot(h0, w_ref[0:HALF, :], preferred_element_type=jnp.float32)
        out = out + jnp.dot(h1, w_ref[HALF:D, :], preferred_element_type=jnp.float32)
    else:
        h = h_ref[...]
        if relu_in:
            h = jnp.maximum(h, 0.0)
        out = jnp.dot(h, w_ref[...], preferred_element_type=jnp.float32)
    y_ref[0] = out[:, 0:HALF]
    y_ref[1] = out[:, HALF:D]
    z_ref[0] = out[:, D:D + HALF] + b_ref[0, 0:HALF]
    z_ref[1] = out[:, D + HALF:2 * D] + b_ref[0, HALF:D]


def _tc_matmul(h, wcat, brel, *, relu_in, split_in, bm=1000):
    grid = (N // bm,)
    if split_in:
        in_spec = pl.BlockSpec((NSC, bm, HALF), lambda i: (0, i, 0))
    else:
        in_spec = pl.BlockSpec((bm, D), lambda i: (i, 0))
    out_spec = pl.BlockSpec((NSC, bm, HALF), lambda i: (0, i, 0))
    return pl.pallas_call(
        functools.partial(_mm_body, relu_in=relu_in, split_in=split_in),
        grid=grid,
        in_specs=[
            in_spec,
            pl.BlockSpec((D, 2 * D), lambda i: (0, 0)),
            pl.BlockSpec((1, D), lambda i: (0, 0)),
        ],
        out_specs=[out_spec, out_spec],
        out_shape=[
            jax.ShapeDtypeStruct((NSC, N, HALF), jnp.float32),
            jax.ShapeDtypeStruct((NSC, N, HALF), jnp.float32),
        ],
    )(h, wcat, brel)


_SC_MESH = plsc.VectorSubcoreMesh(core_axis_name="c", subcore_axis_name="s")


@functools.partial(
    pl.kernel,
    out_type=jax.ShapeDtypeStruct((NSC, N, HALF), jnp.float32),
    mesh=_SC_MESH,
    scratch_types=[
        pltpu.VMEM((NCH, K), jnp.int32),
        pltpu.VMEM((NBUF, K), jnp.int32),
        pltpu.VMEM((NBUF, K), jnp.int32),
        pltpu.VMEM((NBUF, K, HALF), jnp.float32),
        pltpu.VMEM_SHARED((N + NPADROW, HALF), jnp.float32),
        pltpu.SemaphoreType.DMA((NBUF,)),
        pltpu.SemaphoreType.DMA((NBUF,)),
        pltpu.SemaphoreType.DMA,
    ],
)
def _sc_aggregate(y_hbm, z_hbm, packed_hbm, out_hbm,
                  packed_v, src_ring, dst_ring, rows_v, acc, sem, sem_s,
                  sem_i):
    c = lax.axis_index("c")
    s = lax.axis_index("s")
    pltpu.async_copy(z_hbm.at[c, pl.ds(s * RPT, RPT)],
                     acc.at[pl.ds(s * RPT, RPT)], sem_i)

    @pl.when(s == NTILES - 1)
    def _():
        pltpu.async_copy(z_hbm.at[c, pl.ds(RPT * NTILES, REM)],
                         acc.at[pl.ds(RPT * NTILES, REM)], sem_i)

    pltpu.sync_copy(packed_hbm.at[s], packed_v)

    def launch(j):
        b = lax.rem(j, NBUF)
        for t in range(K // 16):
            v = packed_v[j, pl.ds(t * 16, 16)]
            src_ring[b, pl.ds(t * 16, 16)] = jnp.bitwise_and(v, jnp.int32(0xFFFF))
            dst_ring[b, pl.ds(t * 16, 16)] = lax.shift_right_logical(v, jnp.int32(16))
        pltpu.async_copy(y_hbm.at[c].at[src_ring.at[b]], rows_v.at[b], sem.at[b])

    def wait_scatter(j):
        b = lax.rem(j, NBUF)
        pltpu.make_async_copy(rows_v.at[b], acc.at[dst_ring.at[b]],
                              sem_s.at[b]).wait()

    for j0 in range(LOOKAHEAD):
        launch(jnp.int32(j0))

    pltpu.make_async_copy(z_hbm.at[c, pl.ds(s * RPT, RPT)],
                          acc.at[pl.ds(s * RPT, RPT)], sem_i).wait()

    @pl.when(s == NTILES - 1)
    def _():
        pltpu.make_async_copy(z_hbm.at[c, pl.ds(RPT * NTILES, REM)],
                              acc.at[pl.ds(RPT * NTILES, REM)], sem_i).wait()

    plsc.subcore_barrier()

    def body(j, carry):
        @pl.when(j >= NBUF - LOOKAHEAD)
        def _():
            wait_scatter(j - (NBUF - LOOKAHEAD))

        @pl.when(j + LOOKAHEAD < NCH)
        def _():
            launch(j + LOOKAHEAD)
        b = lax.rem(j, NBUF)
        pltpu.make_async_copy(y_hbm.at[c].at[src_ring.at[b]],
                              rows_v.at[b], sem.at[b]).wait()
        pltpu.async_copy(rows_v.at[b], acc.at[dst_ring.at[b]], sem_s.at[b],
                         add=True)
        return carry

    lax.fori_loop(0, NCH, body, 0)
    for d in range(NBUF - LOOKAHEAD):
        wait_scatter(jnp.int32(NCH - 1 - d))
    plsc.subcore_barrier()
    pltpu.sync_copy(acc.at[pl.ds(s * RPT, RPT)], out_hbm.at[c, pl.ds(s * RPT, RPT)])

    @pl.when(s == NTILES - 1)
    def _():
        pltpu.sync_copy(acc.at[pl.ds(RPT * NTILES, REM)],
                        out_hbm.at[c, pl.ds(RPT * NTILES, REM)])


def _pool_body(h_ref, wm_ref, bm_ref, o_ref):
    h0 = jnp.maximum(h_ref[0], 0.0)
    h1 = jnp.maximum(h_ref[1], 0.0)
    p0 = jnp.max(h0, axis=0, keepdims=True)
    p1 = jnp.max(h1, axis=0, keepdims=True)
    pooled = jnp.concatenate([p0, p1], axis=1)
    o_ref[...] = jnp.dot(pooled, wm_ref[...],
                         preferred_element_type=jnp.float32) + bm_ref[...]


def _tc_pool_mlp(h, wmlp_t, bmlp):
    return pl.pallas_call(
        _pool_body,
        out_shape=jax.ShapeDtypeStruct((1, 10), jnp.float32),
    )(h, wmlp_t, bmlp)


def kernel(x, edge_index, Wrel0, brel0, Wroot0, Wrel1, brel1, Wroot1,
           Wrel2, brel2, Wroot2, Wmlp, bmlp):
    src = edge_index[0].reshape(NTILES, EPT)
    dst = edge_index[1].reshape(NTILES, EPT)
    npad = PADE - EPT
    pad_src = jnp.zeros((NTILES, npad), dtype=jnp.int32)
    pad_dst = jnp.broadcast_to(
        N + (jnp.arange(npad, dtype=jnp.int32) % NPADROW), (NTILES, npad))
    src = jnp.concatenate([src, pad_src], axis=1)
    dst = jnp.concatenate([dst, pad_dst], axis=1)
    packed = ((dst << 16) | src).reshape(NTILES, NCH, K)

    def layer(h, Wrel, brel, Wroot, relu_in, split_in):
        wcat = jnp.concatenate([Wrel.T, Wroot.T], axis=1)
        y, z = _tc_matmul(h, wcat, brel.reshape(1, D),
                          relu_in=relu_in, split_in=split_in)
        return _sc_aggregate(y, z, packed)

    h1 = layer(x, Wrel0, brel0, Wroot0, relu_in=False, split_in=False)
    h2 = layer(h1, Wrel1, brel1, Wroot1, relu_in=True, split_in=True)
    h3 = layer(h2, Wrel2, brel2, Wroot2, relu_in=True, split_in=True)
    return _tc_pool_mlp(h3, Wmlp.T, bmlp.reshape(1, 10))
